# Initial kernel scaffold; baseline (speedup 1.0000x reference)
import functools

import jax
import jax.numpy as jnp
from jax import lax
from jax.experimental import pallas as pl
from jax.experimental.pallas import tpu as pltpu
from jax.experimental.pallas import tpu_sc as plsc

N = 10000
E = 320000
FEAT = 128
H1 = 256
H2 = 128
OUT = 64

NC = 2
NS = 16
C = 128
CHUNKS = E // C
CHUNKS_PER_W = -(-CHUNKS // (NC * NS))
ROWS_PER_SUB = N // NS
ZROWS = 125

_HIGH = lax.Precision.HIGHEST


def _dot(a, b):
    return jnp.dot(a, b, precision=_HIGH)


def _gelu(x):
    return 0.5 * x * (1.0 + lax.erf(x * 0.7071067811865476))


def _make_segsum(d, with_deg):
    out_type = [jax.ShapeDtypeStruct((NC, N, d), jnp.float32)]
    if with_deg:
        out_type.append(jax.ShapeDtypeStruct((NC, N, 16), jnp.float32))
    scratch = [
        pltpu.VMEM((C,), jnp.int32),
        pltpu.VMEM((C,), jnp.int32),
        pltpu.VMEM((C, d), jnp.float32),
        pltpu.VMEM((ZROWS, d), jnp.float32),
        pltpu.VMEM_SHARED((N, d), jnp.float32),
        pltpu.SemaphoreType.DMA,
    ]
    if with_deg:
        scratch += [
            pltpu.VMEM((C, 16), jnp.float32),
            pltpu.VMEM((ROWS_PER_SUB, 16), jnp.float32),
            pltpu.VMEM_SHARED((N, 16), jnp.float32),
        ]
    mesh = plsc.VectorSubcoreMesh(core_axis_name="c", subcore_axis_name="s")

    @functools.partial(pl.kernel, mesh=mesh, out_type=out_type,
                       scratch_types=scratch)
    def k(x_hbm, src_hbm, dst_hbm, out_hbm, *rest):
        if with_deg:
            (deg_hbm, idx_s, idx_d, rows, zbuf, acc, sem,
             ones, zbuf16, dacc) = rest
        else:
            idx_s, idx_d, rows, zbuf, acc, sem = rest
        cid = lax.axis_index("c")
        sid = lax.axis_index("s")
        wid = sid * NC + cid

        zero16 = jnp.zeros((16,), jnp.float32)

        @pl.loop(0, ZROWS)
        def _(r):
            for j in range(0, d, 16):
                zbuf[r, pl.ds(j, 16)] = zero16

        base = sid * ROWS_PER_SUB
        for kk in range(ROWS_PER_SUB // ZROWS):
            pltpu.sync_copy(zbuf, acc.at[pl.ds(base + kk * ZROWS, ZROWS)])

        if with_deg:
            one16 = jnp.full((16,), 1.0, jnp.float32)

            @pl.loop(0, C)
            def _(r):
                ones[r, pl.ds(0, 16)] = one16

            @pl.loop(0, ROWS_PER_SUB)
            def _(r):
                zbuf16[r, pl.ds(0, 16)] = zero16

            pltpu.sync_copy(zbuf16, dacc.at[pl.ds(base, ROWS_PER_SUB)])

        plsc.subcore_barrier()

        @pl.loop(0, CHUNKS_PER_W)
        def _(t):
            c = wid + (NC * NS) * t

            @pl.when(c < CHUNKS)
            def _():
                off = c * C
                pltpu.sync_copy(src_hbm.at[pl.ds(off, C)], idx_s)
                pltpu.async_copy(x_hbm.at[idx_s], rows, sem).wait()
                pltpu.sync_copy(dst_hbm.at[pl.ds(off, C)], idx_d)
                pltpu.sync_copy(rows, acc.at[idx_d], add=True)
                if with_deg:
                    pltpu.sync_copy(ones, dacc.at[idx_d], add=True)

        plsc.subcore_barrier()

        for kk in range(ROWS_PER_SUB // ZROWS):
            r0 = base + kk * ZROWS
            pltpu.sync_copy(acc.at[pl.ds(r0, ZROWS)],
                            out_hbm.at[cid, pl.ds(r0, ZROWS)])
        if with_deg:
            pltpu.sync_copy(dacc.at[pl.ds(base, ROWS_PER_SUB)],
                            deg_hbm.at[cid, pl.ds(base, ROWS_PER_SUB)])

    return k


_segsum_deg_128 = _make_segsum(FEAT, True)
_segsum_128 = _make_segsum(H2, False)
_segsum_64 = _make_segsum(OUT, False)


R = 1000
GRID = N // R


def _k1_body(xb, a1b, dgb, Wl1, Wr1, b1, Wl2, Wr2, b2,
             p2o, q2o, invo):
    deg = jnp.maximum(dgb[0] + dgb[1], 1.0)
    inv = 1.0 / deg
    agg = (a1b[0] + a1b[1]) * inv[:, :1]
    h = _gelu(_dot(agg, Wl1[...]) + _dot(xb[...], Wr1[...]) + b1[...])
    p2o[...] = _dot(h, Wl2[...])
    q2o[...] = _dot(h, Wr2[...]) + b2[...]
    invo[...] = inv


def _k2_body(a2b, q2b, invb, Wl3, Wr3, b3, p3o, q3o):
    agg = (a2b[0] + a2b[1]) * invb[:, :1]
    h = _gelu(agg + q2b[...])
    p3o[...] = _dot(h, Wl3[...])
    q3o[...] = _dot(h, Wr3[...]) + b3[...]


def _k3_body(a3b, q3b, invb, w1b, W2, out, accsc):
    i = pl.program_id(0)
    h = (a3b[0] + a3b[1]) * invb[:, :1] + q3b[...]
    m = jnp.max(h, axis=1, keepdims=True)
    lse = jnp.log(jnp.sum(jnp.exp(h - m), axis=1, keepdims=True)) + m
    u = h - lse
    part = jnp.sum(u * w1b[...], axis=0, keepdims=True)

    @pl.when(i == 0)
    def _():
        accsc[...] = part

    @pl.when(i > 0)
    def _():
        accsc[...] += part

    @pl.when(i == pl.num_programs(0) - 1)
    def _():
        out[...] = _dot(accsc[...], W2[...])


def _full(shape):
    nd = len(shape)
    return pl.BlockSpec(shape, lambda i: (0,) * nd)


def _rows(d):
    return pl.BlockSpec((R, d), lambda i: (i, 0))


def _parts(d):
    return pl.BlockSpec((NC, R, d), lambda i: (0, i, 0))


def _k1(x, a1p, dgp, Wl1, Wr1, b1, Wl2, Wr2, b2):
    return pl.pallas_call(
        _k1_body,
        grid=(GRID,),
        in_specs=[_rows(FEAT), _parts(FEAT), _parts(16),
                  _full((FEAT, H1)), _full((FEAT, H1)), _full((H1,)),
                  _full((H1, H2)), _full((H1, H2)), _full((H2,))],
        out_specs=[_rows(H2), _rows(H2), _rows(16)],
        out_shape=[jax.ShapeDtypeStruct((N, H2), jnp.float32),
                   jax.ShapeDtypeStruct((N, H2), jnp.float32),
                   jax.ShapeDtypeStruct((N, 16), jnp.float32)],
    )(x, a1p, dgp, Wl1, Wr1, b1, Wl2, Wr2, b2)


def _k2(a2p, q2, inv, Wl3, Wr3, b3):
    return pl.pallas_call(
        _k2_body,
        grid=(GRID,),
        in_specs=[_parts(H2), _rows(H2), _rows(16),
                  _full((H2, OUT)), _full((H2, OUT)), _full((OUT,))],
        out_specs=[_rows(OUT), _rows(OUT)],
        out_shape=[jax.ShapeDtypeStruct((N, OUT), jnp.float32),
                   jax.ShapeDtypeStruct((N, OUT), jnp.float32)],
    )(a2p, q2, inv, Wl3, Wr3, b3)


def _k3(a3p, q3, inv, w1col, W2):
    return pl.pallas_call(
        _k3_body,
        grid=(GRID,),
        in_specs=[_parts(OUT), _rows(OUT), _rows(16), _rows(1),
                  _full((OUT, OUT))],
        out_specs=[pl.BlockSpec((1, OUT), lambda i: (0, 0))],
        out_shape=jax.ShapeDtypeStruct((1, OUT), jnp.float32),
        scratch_shapes=[pltpu.VMEM((1, OUT), jnp.float32)],
    )(a3p, q3, inv, w1col, W2)


def kernel(features, edges, Wl1, Wr1, b1, Wl2, Wr2, b2, Wl3, Wr3, b3,
           weight1, weight2):
    src = edges[0]
    dst = edges[1]

    a1p, dgp = _segsum_deg_128(features, src, dst)
    p2, q2, inv = _k1(features, a1p, dgp, Wl1, Wr1, b1, Wl2, Wr2, b2)

    a2p = _segsum_128(p2, src, dst)
    p3, q3 = _k2(a2p, q2, inv, Wl3, Wr3, b3)

    a3p = _segsum_64(p3, src, dst)
    w1col = weight1.reshape(N, 1)
    return _k3(a3p, q3, inv, w1col, weight2)


# speedup vs baseline: 6.2705x; 6.2705x over previous
import functools

import jax
import jax.numpy as jnp
from jax import lax
from jax.experimental import pallas as pl
from jax.experimental.pallas import tpu as pltpu
from jax.experimental.pallas import tpu_sc as plsc

N = 10000
E = 320000
FEAT = 128
H1 = 256
H2 = 128
OUT = 64

NC = 2
NS = 16
C = 128
CHUNKS = E // C
CHUNKS_PER_W = -(-CHUNKS // (NC * NS))
BLK = 200
NBLK = N // BLK
BLK_PER_SUB = -(-NBLK // NS)
ZR = 40

_HIGH = lax.Precision.HIGHEST


def _dot(a, b):
    return jnp.dot(a, b, precision=_HIGH)


def _gelu(x):
    return 0.5 * x * (1.0 + lax.erf(x * 0.7071067811865476))


def _make_segsum(d):
    out_type = jax.ShapeDtypeStruct((NC, N, d), jnp.float32)
    scratch = [
        pltpu.VMEM((C,), jnp.int32),
        pltpu.VMEM((C,), jnp.int32),
        pltpu.VMEM((C, d), jnp.float32),
        pltpu.VMEM((ZR, d), jnp.float32),
        pltpu.VMEM_SHARED((N, d), jnp.float32),
        pltpu.SemaphoreType.DMA,
    ]
    mesh = plsc.VectorSubcoreMesh(core_axis_name="c", subcore_axis_name="s")

    @functools.partial(pl.kernel, mesh=mesh, out_type=out_type,
                       scratch_types=scratch)
    def k(x_hbm, src_hbm, dst_hbm, out_hbm, idx_s, idx_d, rows, zbuf, acc,
          sem):
        cid = lax.axis_index("c")
        sid = lax.axis_index("s")
        wid = sid * NC + cid

        zero16 = jnp.zeros((16,), jnp.float32)

        @pl.loop(0, ZR)
        def _(r):
            for j in range(0, d, 16):
                zbuf[r, pl.ds(j, 16)] = zero16

        @pl.loop(0, BLK_PER_SUB)
        def _(t):
            b = sid + NS * t

            @pl.when(b < NBLK)
            def _():
                for kk in range(BLK // ZR):
                    r0 = b * BLK + kk * ZR
                    pltpu.sync_copy(zbuf, acc.at[pl.ds(r0, ZR)])

        plsc.subcore_barrier()

        @pl.loop(0, CHUNKS_PER_W)
        def _(t):
            c = wid + (NC * NS) * t

            @pl.when(c < CHUNKS)
            def _():
                off = c * C
                pltpu.sync_copy(src_hbm.at[pl.ds(off, C)], idx_s)
                pltpu.async_copy(x_hbm.at[idx_s], rows, sem).wait()
                pltpu.sync_copy(dst_hbm.at[pl.ds(off, C)], idx_d)
                pltpu.sync_copy(rows, acc.at[idx_d], add=True)

        plsc.subcore_barrier()

        @pl.loop(0, BLK_PER_SUB)
        def _(t):
            b = sid + NS * t

            @pl.when(b < NBLK)
            def _():
                r0 = b * BLK
                pltpu.sync_copy(acc.at[pl.ds(r0, BLK)],
                                out_hbm.at[cid, pl.ds(r0, BLK)])

    return k


def _make_deg():
    out_type = jax.ShapeDtypeStruct((NC, N, FEAT), jnp.float32)
    scratch = [
        pltpu.VMEM((C,), jnp.int32),
        pltpu.VMEM((C, FEAT), jnp.float32),
        pltpu.VMEM((ZR, FEAT), jnp.float32),
        pltpu.VMEM_SHARED((N, FEAT), jnp.float32),
        pltpu.SemaphoreType.DMA,
    ]
    mesh = plsc.VectorSubcoreMesh(core_axis_name="c", subcore_axis_name="s")

    @functools.partial(pl.kernel, mesh=mesh, out_type=out_type,
                       scratch_types=scratch)
    def k(dst_hbm, deg_hbm, idx_d, ones, zbuf16, dacc, sem):
        cid = lax.axis_index("c")
        sid = lax.axis_index("s")
        wid = sid * NC + cid

        zero16 = jnp.zeros((16,), jnp.float32)
        one16 = jnp.full((16,), 1.0, jnp.float32)

        @pl.loop(0, C)
        def _(r):
            for j in range(0, FEAT, 16):
                ones[r, pl.ds(j, 16)] = one16

        @pl.loop(0, ZR)
        def _(r):
            for j in range(0, FEAT, 16):
                zbuf16[r, pl.ds(j, 16)] = zero16

        @pl.loop(0, BLK_PER_SUB)
        def _(t):
            b = sid + NS * t

            @pl.when(b < NBLK)
            def _():
                for kk in range(BLK // ZR):
                    pltpu.sync_copy(zbuf16,
                                    dacc.at[pl.ds(b * BLK + kk * ZR, ZR)])

        plsc.subcore_barrier()

        @pl.loop(0, CHUNKS_PER_W)
        def _(t):
            c = wid + (NC * NS) * t

            @pl.when(c < CHUNKS)
            def _():
                pltpu.sync_copy(dst_hbm.at[pl.ds(c * C, C)], idx_d)
                pltpu.sync_copy(ones, dacc.at[idx_d], add=True)

        plsc.subcore_barrier()

        @pl.loop(0, BLK_PER_SUB)
        def _(t):
            b = sid + NS * t

            @pl.when(b < NBLK)
            def _():
                r0 = b * BLK
                pltpu.sync_copy(dacc.at[pl.ds(r0, BLK)],
                                deg_hbm.at[cid, pl.ds(r0, BLK)])

    return k


_segsum_128 = _make_segsum(FEAT)
_deg_count = _make_deg()


R = 1000
GRID = N // R


def _k1_body(xb, a1b, dgb, Wl1, Wr1, b1, Wl2, Wr2, b2,
             p2o, q2o, invo):
    deg = jnp.maximum(dgb[0][:, :1] + dgb[1][:, :1], 1.0)
    inv = 1.0 / deg
    agg = (a1b[0] + a1b[1]) * inv
    h = _gelu(_dot(agg, Wl1[...]) + _dot(xb[...], Wr1[...]) + b1[...])
    p2o[...] = _dot(h, Wl2[...])
    q2o[...] = _dot(h, Wr2[...]) + b2[...]
    invo[...] = jnp.broadcast_to(inv, invo.shape)


def _k2_body(a2b, q2b, invb, h2o):
    agg = (a2b[0] + a2b[1]) * invb[:, :1]
    h2o[...] = _gelu(agg + q2b[...])


def _k3_body(a3b, h2b, invb, w1b, Wl3, Wr3, b3, W2, out, accsc):
    i = pl.program_id(0)
    agg = (a3b[0] + a3b[1]) * invb[:, :1]
    h = _dot(agg, Wl3[...]) + _dot(h2b[...], Wr3[...]) + b3[...]
    m = jnp.max(h, axis=1, keepdims=True)
    lse = jnp.log(jnp.sum(jnp.exp(h - m), axis=1, keepdims=True)) + m
    u = h - lse
    part = jnp.sum(u * w1b[...], axis=0, keepdims=True)

    @pl.when(i == 0)
    def _():
        accsc[...] = part

    @pl.when(i > 0)
    def _():
        accsc[...] += part

    @pl.when(i == pl.num_programs(0) - 1)
    def _():
        out[...] = _dot(accsc[...], W2[...])


def _full(shape):
    nd = len(shape)
    return pl.BlockSpec(shape, lambda i: (0,) * nd)


def _rows(d):
    return pl.BlockSpec((R, d), lambda i: (i, 0))


def _parts(d):
    return pl.BlockSpec((NC, R, d), lambda i: (0, i, 0))


def _k1(x, a1p, dgp, Wl1, Wr1, b1, Wl2, Wr2, b2):
    return pl.pallas_call(
        _k1_body,
        grid=(GRID,),
        in_specs=[_rows(FEAT), _parts(FEAT), _parts(FEAT),
                  _full((FEAT, H1)), _full((FEAT, H1)), _full((H1,)),
                  _full((H1, H2)), _full((H1, H2)), _full((H2,))],
        out_specs=[_rows(H2), _rows(H2), _rows(16)],
        out_shape=[jax.ShapeDtypeStruct((N, H2), jnp.float32),
                   jax.ShapeDtypeStruct((N, H2), jnp.float32),
                   jax.ShapeDtypeStruct((N, 16), jnp.float32)],
    )(x, a1p, dgp, Wl1, Wr1, b1, Wl2, Wr2, b2)


def _k2(a2p, q2, inv):
    return pl.pallas_call(
        _k2_body,
        grid=(GRID,),
        in_specs=[_parts(H2), _rows(H2), _rows(16)],
        out_specs=_rows(H2),
        out_shape=jax.ShapeDtypeStruct((N, H2), jnp.float32),
    )(a2p, q2, inv)


def _k3(a3p, h2, inv, w1col, Wl3, Wr3, b3, W2):
    return pl.pallas_call(
        _k3_body,
        grid=(GRID,),
        in_specs=[_parts(H2), _rows(H2), _rows(16), _rows(1),
                  _full((H2, OUT)), _full((H2, OUT)), _full((OUT,)),
                  _full((OUT, OUT))],
        out_specs=pl.BlockSpec((1, OUT), lambda i: (0, 0)),
        out_shape=jax.ShapeDtypeStruct((1, OUT), jnp.float32),
        scratch_shapes=[pltpu.VMEM((1, OUT), jnp.float32)],
    )(a3p, h2, inv, w1col, Wl3, Wr3, b3, W2)


def kernel(features, edges, Wl1, Wr1, b1, Wl2, Wr2, b2, Wl3, Wr3, b3,
           weight1, weight2):
    src = edges[0]
    dst = edges[1]

    dgp = _deg_count(dst)
    a1p = _segsum_128(features, src, dst)
    p2, q2, inv = _k1(features, a1p, dgp, Wl1, Wr1, b1, Wl2, Wr2, b2)

    a2p = _segsum_128(p2, src, dst)
    h2 = _k2(a2p, q2, inv)

    a3p = _segsum_128(h2, src, dst)
    w1col = weight1.reshape(N, 1)
    return _k3(a3p, h2, inv, w1col, Wl3, Wr3, b3, weight2)
